# baseline (speedup 1.0000x reference)
import functools

import jax
import jax.numpy as jnp
import numpy as np
from jax import lax
from jax.experimental import pallas as pl
from jax.experimental.pallas import tpu as pltpu
from jax.experimental.pallas import tpu_sc as plsc

_NC = 2
_NS = 16
_NW = _NC * _NS
_LANES = 16


def _rotl32(x, r):
    return ((x << np.uint32(r)) | (x >> np.uint32(32 - r))).astype(np.uint32)


def _threefry2x32(k0, k1, x0, x1):
    k0 = np.uint32(k0)
    k1 = np.uint32(k1)
    k2 = np.uint32(k0 ^ k1 ^ np.uint32(0x1BD11BDA))
    ks = (k0, k1, k2)
    x0 = (x0.astype(np.uint32) + k0).astype(np.uint32)
    x1 = (x1.astype(np.uint32) + k1).astype(np.uint32)
    for r in range(5):
        for rot in ((13, 15, 26, 6) if r % 2 == 0 else (17, 29, 16, 24)):
            x0 = (x0 + x1).astype(np.uint32)
            x1 = _rotl32(x1, rot)
            x1 = (x0 ^ x1).astype(np.uint32)
        x0 = (x0 + ks[(r + 1) % 3]).astype(np.uint32)
        x1 = (x1 + ks[(r + 2) % 3] + np.uint32(r + 1)).astype(np.uint32)
    return x0, x1


_GUM_CACHE = {}


def _gumbel_const(e):
    if e not in _GUM_CACHE:
        i = np.arange(e, dtype=np.uint64)
        hi = (i >> np.uint64(32)).astype(np.uint32)
        lo = (i & np.uint64(0xFFFFFFFF)).astype(np.uint32)
        b0, b1 = _threefry2x32(0, 12345, hi, lo)
        bits = b0 ^ b1
        u = ((bits >> np.uint32(9)) | np.uint32(0x3F800000)).view(np.float32) \
            - np.float32(1.0)
        bias = np.float32(0.0001)
        one = np.float32(1.0)
        eps = (bias - (one - bias)) * u + (one - bias)
        _GUM_CACHE[e] = np.log(eps) - np.log(one - eps)
    return _GUM_CACHE[e]


def _node_mlp_body(x_ref, ws1t_ref, bs1_ref, ws2t_ref, bs2_ref,
                   wd1t_ref, bd1_ref, wd2t_ref, bd2_ref, o_ref):
    hs = lax.dot_general(ws1t_ref[...], x_ref[...], (((1,), (1,)), ((), ())),
                         preferred_element_type=jnp.float32)
    hs = jnp.maximum(hs + bs1_ref[...], 0.0)
    o_ref[0:1, :] = lax.dot_general(
        ws2t_ref[...], hs, (((1,), (0,)), ((), ())),
        preferred_element_type=jnp.float32) + bs2_ref[...]
    hd = lax.dot_general(wd1t_ref[...], x_ref[...], (((1,), (1,)), ((), ())),
                         preferred_element_type=jnp.float32)
    hd = jnp.maximum(hd + bd1_ref[...], 0.0)
    o_ref[1:2, :] = lax.dot_general(
        wd2t_ref[...], hd, (((1,), (0,)), ((), ())),
        preferred_element_type=jnp.float32) + bd2_ref[...]


def _node_tables(node_emb, ws1t, bs1, ws2t, bs2, wd1t, bd1, wd2t, bd2):
    n, d = node_emb.shape
    return pl.pallas_call(
        _node_mlp_body,
        out_shape=jax.ShapeDtypeStruct((2, n), jnp.float32),
    )(node_emb, ws1t, bs1, ws2t, bs2, wd1t, bd1, wd2t, bd2)


def _edge_mlp_body(inv_temp, blk, xt_ref, w1t_ref, b1_ref, w2t_ref, b2_ref,
                   g_ref, ws_hbm, o_ref, p_ref, ws_v, sem):
    i = pl.program_id(0)
    cp = pltpu.make_async_copy(ws_hbm.at[pl.ds(i * blk, blk)], ws_v, sem)
    cp.start()
    ht = lax.dot_general(w1t_ref[...], xt_ref[...], (((1,), (0,)), ((), ())),
                         precision=lax.Precision.DEFAULT,
                         preferred_element_type=jnp.float32)
    ht = jnp.maximum(ht + b1_ref[...], 0.0)
    row = (lax.dot_general(w2t_ref[...], ht, (((1,), (0,)), ((), ())),
                           precision=lax.Precision.DEFAULT,
                           preferred_element_type=jnp.float32)
           + b2_ref[...])
    cp.wait()
    wsv = ws_v[...].reshape(1, 1, blk)
    x = (row.reshape(1, 1, row.shape[1]) + g_ref[...] + wsv) * inv_temp
    sig = 1.0 / (1.0 + jnp.exp(-x))
    o_ref[...] = sig
    p_ref[...] = jnp.broadcast_to(jnp.sum(sig), p_ref.shape)


_EDGE_BLK = 12800


def _edge_finale(relation_t, w1t, b1col, w2t, b2, gum3, wsum, inv_temp):
    de, e = relation_t.shape
    blk = _EDGE_BLK
    assert e % blk == 0
    grid = e // blk
    h = w1t.shape[0]
    return pl.pallas_call(
        functools.partial(_edge_mlp_body, inv_temp, blk),
        grid=(grid,),
        in_specs=[
            pl.BlockSpec((de, blk), lambda i: (0, i)),
            pl.BlockSpec((h, de), lambda i: (0, 0)),
            pl.BlockSpec((h, 1), lambda i: (0, 0)),
            pl.BlockSpec((1, h), lambda i: (0, 0)),
            pl.BlockSpec((1, 1), lambda i: (0, 0)),
            pl.BlockSpec((1, 1, blk), lambda i: (i, 0, 0)),
            pl.BlockSpec(memory_space=pl.ANY),
        ],
        out_specs=[
            pl.BlockSpec((1, 1, blk), lambda i: (i, 0, 0)),
            pl.BlockSpec((1, 1, 128), lambda i: (i, 0, 0)),
        ],
        out_shape=[
            jax.ShapeDtypeStruct((grid, 1, blk), jnp.float32),
            jax.ShapeDtypeStruct((grid, 1, 128), jnp.float32),
        ],
        scratch_shapes=[
            pltpu.VMEM((blk,), jnp.float32),
            pltpu.SemaphoreType.DMA,
        ],
    )(relation_t, w1t, b1col, w2t, b2, gum3, wsum)


_ALIGN = 128


def _sc_gather(wtab2, edge_index, n):
    e = edge_index.shape[1]
    assert e % (_NW * _LANES) == 0
    chunk = e // _NW
    nvec = chunk // _LANES
    asz = -(-chunk // _ALIGN) * _ALIGN
    mesh = plsc.VectorSubcoreMesh(core_axis_name="c", subcore_axis_name="s")

    @functools.partial(
        pl.kernel,
        out_type=jax.ShapeDtypeStruct((e,), jnp.float32),
        mesh=mesh,
        compiler_params=pltpu.CompilerParams(needs_layout_passes=False),
        scratch_types=[
            pltpu.VMEM((2, n), jnp.float32),
            pltpu.VMEM((2, asz), jnp.int32),
            pltpu.VMEM((chunk,), jnp.float32),
        ],
    )
    def sc_k(wtab_hbm, eidx_hbm, out_hbm, wtab_v, eidx_v, out_v):
        wid = lax.axis_index("s") * _NC + lax.axis_index("c")
        base = wid * chunk
        off = lax.rem(base, _ALIGN)
        abase = pl.multiple_of(base - off, _ALIGN)
        pltpu.sync_copy(wtab_hbm, wtab_v)
        pltpu.sync_copy(eidx_hbm.at[:, pl.ds(abase, asz)], eidx_v)
        zero = jnp.zeros((_LANES,), jnp.int32)
        one = zero + 1

        def body(i, carry):
            o = off + i * _LANES
            sidx = eidx_v[0, pl.ds(o, _LANES)]
            didx = eidx_v[1, pl.ds(o, _LANES)]
            gs = plsc.load_gather(wtab_v, [zero, sidx])
            gd = plsc.load_gather(wtab_v, [one, didx])
            out_v[pl.ds(i * _LANES, _LANES)] = gs + gd
            return carry

        lax.fori_loop(0, nvec, body, 0)
        pltpu.sync_copy(out_v, out_hbm.at[pl.ds(base, chunk)])

    return sc_k(wtab2, edge_index)


def kernel(node_emb, edge_index, relation_emb, Ws1, bs1, Ws2, bs2,
           Wd1, bd1, Wd2, bd2, We1, be1, We2, be2):
    n, d = node_emb.shape
    e = edge_index.shape[1]
    h = Ws1.shape[1]
    temperature = 0.5
    blk = _EDGE_BLK
    grid = e // blk

    gum3 = jnp.asarray(_gumbel_const(e).reshape(grid, 1, blk))

    wtab2 = _node_tables(node_emb, Ws1.T, bs1.reshape(h, 1), Ws2.T,
                         bs2.reshape(1, 1), Wd1.T, bd1.reshape(h, 1),
                         Wd2.T, bd2.reshape(1, 1))
    wsum = _sc_gather(wtab2, edge_index, n)

    out3, parts = _edge_finale(
        relation_emb.T, We1.T, be1.reshape(h, 1), We2.T, be2.reshape(1, 1),
        gum3, wsum, inv_temp=1.0 / temperature)

    reg = 1.0 - parts[:, 0, 0].sum() / e
    return (reg, out3.reshape(e, 1, 1))


# speedup vs baseline: 1.0742x; 1.0742x over previous
import functools

import jax
import jax.numpy as jnp
import numpy as np
from jax import lax
from jax.experimental import pallas as pl
from jax.experimental.pallas import tpu as pltpu
from jax.experimental.pallas import tpu_sc as plsc

_NC = 2
_NS = 16
_NW = _NC * _NS
_LANES = 16


def _rotl32(x, r):
    return ((x << np.uint32(r)) | (x >> np.uint32(32 - r))).astype(np.uint32)


def _threefry2x32(k0, k1, x0, x1):
    k0 = np.uint32(k0)
    k1 = np.uint32(k1)
    k2 = np.uint32(k0 ^ k1 ^ np.uint32(0x1BD11BDA))
    ks = (k0, k1, k2)
    x0 = (x0.astype(np.uint32) + k0).astype(np.uint32)
    x1 = (x1.astype(np.uint32) + k1).astype(np.uint32)
    for r in range(5):
        for rot in ((13, 15, 26, 6) if r % 2 == 0 else (17, 29, 16, 24)):
            x0 = (x0 + x1).astype(np.uint32)
            x1 = _rotl32(x1, rot)
            x1 = (x0 ^ x1).astype(np.uint32)
        x0 = (x0 + ks[(r + 1) % 3]).astype(np.uint32)
        x1 = (x1 + ks[(r + 2) % 3] + np.uint32(r + 1)).astype(np.uint32)
    return x0, x1


_GUM_CACHE = {}


def _gumbel_const(e):
    if e not in _GUM_CACHE:
        i = np.arange(e, dtype=np.uint64)
        hi = (i >> np.uint64(32)).astype(np.uint32)
        lo = (i & np.uint64(0xFFFFFFFF)).astype(np.uint32)
        b0, b1 = _threefry2x32(0, 12345, hi, lo)
        bits = b0 ^ b1
        u = ((bits >> np.uint32(9)) | np.uint32(0x3F800000)).view(np.float32) \
            - np.float32(1.0)
        bias = np.float32(0.0001)
        one = np.float32(1.0)
        eps = (bias - (one - bias)) * u + (one - bias)
        _GUM_CACHE[e] = np.log(eps) - np.log(one - eps)
    return _GUM_CACHE[e]


def _node_mlp_body(x_ref, ws1t_ref, bs1_ref, ws2t_ref, bs2_ref,
                   wd1t_ref, bd1_ref, wd2t_ref, bd2_ref, o_ref):
    hs = lax.dot_general(ws1t_ref[...], x_ref[...], (((1,), (1,)), ((), ())),
                         preferred_element_type=jnp.float32)
    hs = jnp.maximum(hs + bs1_ref[...], 0.0)
    o_ref[0:1, :] = lax.dot_general(
        ws2t_ref[...], hs, (((1,), (0,)), ((), ())),
        preferred_element_type=jnp.float32) + bs2_ref[...]
    hd = lax.dot_general(wd1t_ref[...], x_ref[...], (((1,), (1,)), ((), ())),
                         preferred_element_type=jnp.float32)
    hd = jnp.maximum(hd + bd1_ref[...], 0.0)
    o_ref[1:2, :] = lax.dot_general(
        wd2t_ref[...], hd, (((1,), (0,)), ((), ())),
        preferred_element_type=jnp.float32) + bd2_ref[...]


def _node_tables(node_emb, ws1t, bs1, ws2t, bs2, wd1t, bd1, wd2t, bd2):
    n, d = node_emb.shape
    return pl.pallas_call(
        _node_mlp_body,
        out_shape=jax.ShapeDtypeStruct((2, n), jnp.float32),
    )(node_emb, ws1t, bs1, ws2t, bs2, wd1t, bd1, wd2t, bd2)


def _edge_mlp_body(inv_temp, blk, xt_ref, w1t_ref, b1_ref, w2t_ref, b2_ref,
                   g_ref, ws_ref, o_ref, p_ref):
    ht = lax.dot_general(w1t_ref[...], xt_ref[...], (((1,), (0,)), ((), ())),
                         precision=lax.Precision.DEFAULT,
                         preferred_element_type=jnp.float32)
    ht = jnp.maximum(ht + b1_ref[...], 0.0)
    row = (lax.dot_general(w2t_ref[...], ht, (((1,), (0,)), ((), ())),
                           precision=lax.Precision.DEFAULT,
                           preferred_element_type=jnp.float32)
           + b2_ref[...])
    x = (row.reshape(1, 1, row.shape[1]) + g_ref[...] + ws_ref[...]) * inv_temp
    sig = 1.0 / (1.0 + jnp.exp(-x))
    o_ref[...] = sig
    p_ref[...] = jnp.broadcast_to(jnp.sum(sig), p_ref.shape)


_EDGE_BLK = 12800


def _edge_finale(relation_t, w1t, b1col, w2t, b2, gum3, wsum3, inv_temp):
    de, e = relation_t.shape
    blk = _EDGE_BLK
    assert e % blk == 0
    grid = e // blk
    h = w1t.shape[0]
    return pl.pallas_call(
        functools.partial(_edge_mlp_body, inv_temp, blk),
        grid=(grid,),
        in_specs=[
            pl.BlockSpec((de, blk), lambda i: (0, i)),
            pl.BlockSpec((h, de), lambda i: (0, 0)),
            pl.BlockSpec((h, 1), lambda i: (0, 0)),
            pl.BlockSpec((1, h), lambda i: (0, 0)),
            pl.BlockSpec((1, 1), lambda i: (0, 0)),
            pl.BlockSpec((1, 1, blk), lambda i: (i, 0, 0)),
            pl.BlockSpec((1, 1, blk), lambda i: (i, 0, 0)),
        ],
        out_specs=[
            pl.BlockSpec((1, 1, blk), lambda i: (i, 0, 0)),
            pl.BlockSpec((1, 1, 128), lambda i: (i, 0, 0)),
        ],
        out_shape=[
            jax.ShapeDtypeStruct((grid, 1, blk), jnp.float32),
            jax.ShapeDtypeStruct((grid, 1, 128), jnp.float32),
        ],
    )(relation_t, w1t, b1col, w2t, b2, gum3, wsum3)


_ALIGN = 128


def _sc_gather(wtab2, edge_index, n):
    e = edge_index.shape[1]
    assert e % (_NW * _LANES) == 0
    chunk = e // _NW
    nvec = chunk // _LANES
    asz = -(-chunk // _ALIGN) * _ALIGN
    mesh = plsc.VectorSubcoreMesh(core_axis_name="c", subcore_axis_name="s")

    @functools.partial(
        pl.kernel,
        out_type=jax.ShapeDtypeStruct((e,), jnp.float32),
        mesh=mesh,
        compiler_params=pltpu.CompilerParams(needs_layout_passes=False),
        scratch_types=[
            pltpu.VMEM((2, n), jnp.float32),
            pltpu.VMEM((2, asz), jnp.int32),
            pltpu.VMEM((chunk,), jnp.float32),
        ],
    )
    def sc_k(wtab_hbm, eidx_hbm, out_hbm, wtab_v, eidx_v, out_v):
        wid = lax.axis_index("s") * _NC + lax.axis_index("c")
        base = wid * chunk
        off = lax.rem(base, _ALIGN)
        abase = pl.multiple_of(base - off, _ALIGN)
        pltpu.sync_copy(wtab_hbm, wtab_v)
        pltpu.sync_copy(eidx_hbm.at[:, pl.ds(abase, asz)], eidx_v)
        zero = jnp.zeros((_LANES,), jnp.int32)
        one = zero + 1

        def body(i, carry):
            o = off + i * _LANES
            sidx = eidx_v[0, pl.ds(o, _LANES)]
            didx = eidx_v[1, pl.ds(o, _LANES)]
            gs = plsc.load_gather(wtab_v, [zero, sidx])
            gd = plsc.load_gather(wtab_v, [one, didx])
            out_v[pl.ds(i * _LANES, _LANES)] = gs + gd
            return carry

        lax.fori_loop(0, nvec, body, 0)
        pltpu.sync_copy(out_v, out_hbm.at[pl.ds(base, chunk)])

    return sc_k(wtab2, edge_index)


def kernel(node_emb, edge_index, relation_emb, Ws1, bs1, Ws2, bs2,
           Wd1, bd1, Wd2, bd2, We1, be1, We2, be2):
    n, d = node_emb.shape
    e = edge_index.shape[1]
    h = Ws1.shape[1]
    temperature = 0.5
    blk = _EDGE_BLK
    grid = e // blk

    gum3 = jnp.asarray(_gumbel_const(e).reshape(grid, 1, blk))

    wtab2 = _node_tables(node_emb, Ws1.T, bs1.reshape(h, 1), Ws2.T,
                         bs2.reshape(1, 1), Wd1.T, bd1.reshape(h, 1),
                         Wd2.T, bd2.reshape(1, 1))
    wsum = _sc_gather(wtab2, edge_index, n)

    out3, parts = _edge_finale(
        relation_emb.T, We1.T, be1.reshape(h, 1), We2.T, be2.reshape(1, 1),
        gum3, wsum.reshape(grid, 1, blk), inv_temp=1.0 / temperature)

    reg = 1.0 - parts[:, 0, 0].sum() / e
    return (reg, out3.reshape(e, 1, 1))


# speedup vs baseline: 1.2099x; 1.1263x over previous
import functools

import jax
import jax.numpy as jnp
import numpy as np
from jax import lax
from jax.experimental import pallas as pl
from jax.experimental.pallas import tpu as pltpu
from jax.experimental.pallas import tpu_sc as plsc

_NC = 2
_NS = 16
_NW = _NC * _NS
_LANES = 16


def _rotl32(x, r):
    return ((x << np.uint32(r)) | (x >> np.uint32(32 - r))).astype(np.uint32)


def _threefry2x32(k0, k1, x0, x1):
    k0 = np.uint32(k0)
    k1 = np.uint32(k1)
    k2 = np.uint32(k0 ^ k1 ^ np.uint32(0x1BD11BDA))
    ks = (k0, k1, k2)
    x0 = (x0.astype(np.uint32) + k0).astype(np.uint32)
    x1 = (x1.astype(np.uint32) + k1).astype(np.uint32)
    for r in range(5):
        for rot in ((13, 15, 26, 6) if r % 2 == 0 else (17, 29, 16, 24)):
            x0 = (x0 + x1).astype(np.uint32)
            x1 = _rotl32(x1, rot)
            x1 = (x0 ^ x1).astype(np.uint32)
        x0 = (x0 + ks[(r + 1) % 3]).astype(np.uint32)
        x1 = (x1 + ks[(r + 2) % 3] + np.uint32(r + 1)).astype(np.uint32)
    return x0, x1


_GUM_CACHE = {}


def _gumbel_const(e):
    if e not in _GUM_CACHE:
        i = np.arange(e, dtype=np.uint64)
        hi = (i >> np.uint64(32)).astype(np.uint32)
        lo = (i & np.uint64(0xFFFFFFFF)).astype(np.uint32)
        b0, b1 = _threefry2x32(0, 12345, hi, lo)
        bits = b0 ^ b1
        u = ((bits >> np.uint32(9)) | np.uint32(0x3F800000)).view(np.float32) \
            - np.float32(1.0)
        bias = np.float32(0.0001)
        one = np.float32(1.0)
        eps = (bias - (one - bias)) * u + (one - bias)
        _GUM_CACHE[e] = np.log(eps) - np.log(one - eps)
    return _GUM_CACHE[e]


def _node_mlp_body(x_ref, ws1t_ref, bs1_ref, ws2t_ref, bs2_ref,
                   wd1t_ref, bd1_ref, wd2t_ref, bd2_ref, o_ref):
    hs = lax.dot_general(ws1t_ref[...], x_ref[...], (((1,), (1,)), ((), ())),
                         preferred_element_type=jnp.float32)
    hs = jnp.maximum(hs + bs1_ref[...], 0.0)
    o_ref[0:1, :] = lax.dot_general(
        ws2t_ref[...], hs, (((1,), (0,)), ((), ())),
        preferred_element_type=jnp.float32) + bs2_ref[...]
    hd = lax.dot_general(wd1t_ref[...], x_ref[...], (((1,), (1,)), ((), ())),
                         preferred_element_type=jnp.float32)
    hd = jnp.maximum(hd + bd1_ref[...], 0.0)
    o_ref[1:2, :] = lax.dot_general(
        wd2t_ref[...], hd, (((1,), (0,)), ((), ())),
        preferred_element_type=jnp.float32) + bd2_ref[...]


def _node_tables(node_emb, ws1t, bs1, ws2t, bs2, wd1t, bd1, wd2t, bd2):
    n, d = node_emb.shape
    return pl.pallas_call(
        _node_mlp_body,
        out_shape=jax.ShapeDtypeStruct((2, n), jnp.float32),
    )(node_emb, ws1t, bs1, ws2t, bs2, wd1t, bd1, wd2t, bd2)


def _edge_mlp_body(inv_temp, blk, xt_ref, w1t_ref, b1_ref, w2t_ref, b2_ref,
                   g_ref, ws_ref, o_ref, p_ref):
    ht = lax.dot_general(w1t_ref[...], xt_ref[...], (((1,), (0,)), ((), ())),
                         precision=lax.Precision.DEFAULT,
                         preferred_element_type=jnp.float32)
    ht = jnp.maximum(ht + b1_ref[...], 0.0)
    row = (lax.dot_general(w2t_ref[...], ht, (((1,), (0,)), ((), ())),
                           precision=lax.Precision.DEFAULT,
                           preferred_element_type=jnp.float32)
           + b2_ref[...])
    x = (row.reshape(1, 1, row.shape[1]) + g_ref[...] + ws_ref[...]) * inv_temp
    sig = 1.0 / (1.0 + jnp.exp(-x))
    o_ref[...] = sig
    p_ref[...] = jnp.broadcast_to(jnp.sum(sig), p_ref.shape)


_EDGE_BLK = 32000


def _edge_finale(relation_t, w1t, b1col, w2t, b2, gum3, wsum3, inv_temp):
    de, e = relation_t.shape
    blk = _EDGE_BLK
    assert e % blk == 0
    grid = e // blk
    h = w1t.shape[0]
    return pl.pallas_call(
        functools.partial(_edge_mlp_body, inv_temp, blk),
        grid=(grid,),
        in_specs=[
            pl.BlockSpec((de, blk), lambda i: (0, i)),
            pl.BlockSpec((h, de), lambda i: (0, 0)),
            pl.BlockSpec((h, 1), lambda i: (0, 0)),
            pl.BlockSpec((1, h), lambda i: (0, 0)),
            pl.BlockSpec((1, 1), lambda i: (0, 0)),
            pl.BlockSpec((1, 1, blk), lambda i: (i, 0, 0)),
            pl.BlockSpec((1, 1, blk), lambda i: (i, 0, 0)),
        ],
        out_specs=[
            pl.BlockSpec((1, 1, blk), lambda i: (i, 0, 0)),
            pl.BlockSpec((1, 1, 128), lambda i: (i, 0, 0)),
        ],
        out_shape=[
            jax.ShapeDtypeStruct((grid, 1, blk), jnp.float32),
            jax.ShapeDtypeStruct((grid, 1, 128), jnp.float32),
        ],
    )(relation_t, w1t, b1col, w2t, b2, gum3, wsum3)


_ALIGN = 128


def _sc_gather(wtab2, edge_index, n):
    e = edge_index.shape[1]
    assert e % (_NW * _LANES) == 0
    chunk = e // _NW
    nvec = chunk // _LANES
    asz = -(-chunk // _ALIGN) * _ALIGN
    mesh = plsc.VectorSubcoreMesh(core_axis_name="c", subcore_axis_name="s")

    @functools.partial(
        pl.kernel,
        out_type=jax.ShapeDtypeStruct((e,), jnp.float32),
        mesh=mesh,
        compiler_params=pltpu.CompilerParams(needs_layout_passes=False),
        scratch_types=[
            pltpu.VMEM((2, n), jnp.float32),
            pltpu.VMEM((2, asz), jnp.int32),
            pltpu.VMEM((chunk,), jnp.float32),
        ],
    )
    def sc_k(wtab_hbm, eidx_hbm, out_hbm, wtab_v, eidx_v, out_v):
        wid = lax.axis_index("s") * _NC + lax.axis_index("c")
        base = wid * chunk
        off = lax.rem(base, _ALIGN)
        abase = pl.multiple_of(base - off, _ALIGN)
        pltpu.sync_copy(wtab_hbm, wtab_v)
        pltpu.sync_copy(eidx_hbm.at[:, pl.ds(abase, asz)], eidx_v)
        zero = jnp.zeros((_LANES,), jnp.int32)
        one = zero + 1

        def body(i, carry):
            o = off + i * _LANES
            sidx = eidx_v[0, pl.ds(o, _LANES)]
            didx = eidx_v[1, pl.ds(o, _LANES)]
            gs = plsc.load_gather(wtab_v, [zero, sidx])
            gd = plsc.load_gather(wtab_v, [one, didx])
            out_v[pl.ds(i * _LANES, _LANES)] = gs + gd
            return carry

        lax.fori_loop(0, nvec, body, 0)
        pltpu.sync_copy(out_v, out_hbm.at[pl.ds(base, chunk)])

    return sc_k(wtab2, edge_index)


def kernel(node_emb, edge_index, relation_emb, Ws1, bs1, Ws2, bs2,
           Wd1, bd1, Wd2, bd2, We1, be1, We2, be2):
    n, d = node_emb.shape
    e = edge_index.shape[1]
    h = Ws1.shape[1]
    temperature = 0.5
    blk = _EDGE_BLK
    grid = e // blk

    gum3 = jnp.asarray(_gumbel_const(e).reshape(grid, 1, blk))

    wtab2 = _node_tables(node_emb, Ws1.T, bs1.reshape(h, 1), Ws2.T,
                         bs2.reshape(1, 1), Wd1.T, bd1.reshape(h, 1),
                         Wd2.T, bd2.reshape(1, 1))
    wsum = _sc_gather(wtab2, edge_index, n)

    out3, parts = _edge_finale(
        relation_emb.T, We1.T, be1.reshape(h, 1), We2.T, be2.reshape(1, 1),
        gum3, wsum.reshape(grid, 1, blk), inv_temp=1.0 / temperature)

    reg = 1.0 - parts[:, 0, 0].sum() / e
    return (reg, out3.reshape(e, 1, 1))
